# baseline (speedup 1.0000x reference)
import functools

import jax
import jax.numpy as jnp
from jax import lax
from jax.experimental import pallas as pl
from jax.experimental.pallas import tpu as pltpu
from jax.experimental.pallas import tpu_sc as plsc

B = 4096
L = 200
D = 64
V = 1000
NC = 2
NS = 16
NW = NC * NS
LT = L // 8
BT = B // 128
PH = 2
DH = D // PH


def _body(ridx4, cidx4, rowt_hbm, colt_hbm, out5,
          tstage, rowT_v, colT_v, rtile, ctile, obuf, semout):
    bt = lax.axis_index("s") * NC + lax.axis_index("c")
    lanes = lax.iota(jnp.int32, 16)

    def load_table_half(tbl_hbm, dstT, ph):
        pltpu.sync_copy(tbl_hbm.at[pl.ds(0, V), pl.ds(ph * DH, DH)], tstage)

        def trow(e, carry):
            ev = jnp.full((16,), e, jnp.int32)
            for g in range(DH // 16):
                v = tstage[e, pl.ds(g * 16, 16)]
                plsc.store_scatter(dstT, [lanes + (g * 16), ev], v)
            return carry
        lax.fori_loop(0, V, trow, 0)

    def out_cp(l, ph, p):
        return pltpu.make_async_copy(
            obuf.at[p],
            out5.at[l, pl.ds(ph * (DH // 8), DH // 8), bt], semout)

    for ph in range(PH):
        load_table_half(rowt_hbm, rowT_v, ph)
        load_table_half(colt_hbm, colT_v, ph)

        def ltile(tr, carry):
            pltpu.sync_copy(ridx4.at[tr, bt], rtile)
            pltpu.sync_copy(cidx4.at[tr, bt], ctile)

            def lrow(r, c2):
                l = tr * 8 + r
                p = r & 1

                @pl.when(l >= 2)
                def _drain_prev():
                    out_cp(0, 0, 0).wait()

                for g in range(8):
                    i16r = rtile[r, pl.ds(g * 16, 16)]
                    i16c = ctile[r, pl.ds(g * 16, 16)]
                    for dl in range(DH):
                        dv = jnp.full((16,), dl, jnp.int32)
                        v = (plsc.load_gather(rowT_v, [dv, i16r]) +
                             plsc.load_gather(colT_v, [dv, i16c]))
                        obuf[p, dl // 8, dl % 8, pl.ds(g * 16, 16)] = v
                out_cp(l, ph, p).start()
                return c2

            lax.fori_loop(0, 8, lrow, 0)
            return carry

        lax.fori_loop(0, LT, ltile, 0)
        out_cp(0, 0, 0).wait()
        out_cp(0, 0, 0).wait()


@jax.jit
def kernel(row_indices, col_indices, row_table, col_table):
    ridx4 = jnp.transpose(
        row_indices.astype(jnp.int32).T.reshape(LT, 8, BT, 128), (0, 2, 1, 3))
    cidx4 = jnp.transpose(
        col_indices.astype(jnp.int32).T.reshape(LT, 8, BT, 128), (0, 2, 1, 3))
    k = pl.kernel(
        _body,
        mesh=plsc.VectorSubcoreMesh(core_axis_name="c", subcore_axis_name="s"),
        compiler_params=pltpu.CompilerParams(
            use_tc_tiling_on_sc=False, needs_layout_passes=False),
        out_type=jax.ShapeDtypeStruct((L, D // 8, BT, 8, 128), jnp.float32),
        scratch_types=[
            pltpu.VMEM((V, DH), jnp.float32),
            pltpu.VMEM((DH, V), jnp.float32),
            pltpu.VMEM((DH, V), jnp.float32),
            pltpu.VMEM((8, 128), jnp.int32),
            pltpu.VMEM((8, 128), jnp.int32),
            pltpu.VMEM((2, DH // 8, 8, 128), jnp.float32),
            pltpu.SemaphoreType.DMA,
        ],
    )
    out5 = k(ridx4, cidx4, row_table, col_table)
    return out5.transpose(2, 4, 0, 1, 3).reshape(B, L, D)


# speedup vs baseline: 1.7359x; 1.7359x over previous
import functools

import jax
import jax.numpy as jnp
from jax import lax
from jax.experimental import pallas as pl
from jax.experimental.pallas import tpu as pltpu
from jax.experimental.pallas import tpu_sc as plsc

B = 4096
L = 200
D = 64
N = B * L
V = 1000
NC = 2
NS = 16
NW = NC * NS
LT = L // 8
BT = B // 128
BAND = B // NW
PH = 2
BPH = BAND // PH
RING = 4
L0 = 128
L1 = L - L0


def _body(ridx4, cidx4, rowt_hbm, colt_hbm, out_hbm,
          rblk, cblk, rT, cT, outbuf, rowt_sh, colt_sh,
          semr, semc, semout):
    sid = lax.axis_index("s")
    bt = sid * NC + lax.axis_index("c")
    band0 = bt * BAND

    @pl.when(sid == 0)
    def _stage_tables():
        pltpu.sync_copy(rowt_hbm, rowt_sh)
        pltpu.sync_copy(colt_hbm, colt_sh)
    plsc.subcore_barrier()

    lanes = lax.iota(jnp.int32, 16)

    def transpose_block(src, dst):
        def ttile(tr, carry):
            for r in range(8):
                lv = jnp.full((16,), tr * 8 + r, jnp.int32)
                for g in range(BPH // 16):
                    v = src[tr, r, pl.ds(g * 16, 16)]
                    plsc.store_scatter(dst, [lanes + (g * 16), lv], v)
            return carry
        lax.fori_loop(0, LT, ttile, 0)

    def row_cps(b, s):
        return [
            pltpu.make_async_copy(
                rowt_sh.at[rT.at[b, pl.ds(0, L0)]],
                outbuf.at[pl.ds(s * L, L0)], semr),
            pltpu.make_async_copy(
                rowt_sh.at[rT.at[b, pl.ds(L0, L1)]],
                outbuf.at[pl.ds(s * L + L0, L1)], semr),
        ]

    def col_cps(b, s):
        return [
            pltpu.make_async_copy(
                colt_sh.at[cT.at[b, pl.ds(0, L0)]],
                outbuf.at[pl.ds(s * L, L0)], semc),
            pltpu.make_async_copy(
                colt_sh.at[cT.at[b, pl.ds(L0, L1)]],
                outbuf.at[pl.ds(s * L + L0, L1)], semc),
        ]

    def out_cp(bg, s):
        return pltpu.make_async_copy(
            outbuf.at[pl.ds(s * L, L)],
            out_hbm.at[pl.ds((band0 + bg) * L, L)], semout)

    def fire(cps, **kw):
        for cp in cps:
            cp.start(**kw)

    def drain(cps):
        for cp in cps:
            cp.wait()

    for ph in range(PH):
        c0 = ph * BPH
        pltpu.sync_copy(
            ridx4.at[pl.ds(0, LT), bt, pl.ds(0, 8), pl.ds(c0, BPH)], rblk)
        pltpu.sync_copy(
            cidx4.at[pl.ds(0, LT), bt, pl.ds(0, 8), pl.ds(c0, BPH)], cblk)
        transpose_block(rblk, rT)
        transpose_block(cblk, cT)

        def step(b, fire_next, drain_out):
            s = b & (RING - 1)
            drain(row_cps(b, s))
            fire(col_cps(b, s), add=True)
            if drain_out:
                drain([out_cp(0, s)])
            if fire_next:
                fire(row_cps(b + 1, (b + 1) & (RING - 1)))
            drain(col_cps(b, s))
            fire([out_cp(c0 + b, s)])

        fire(row_cps(0, 0))
        step(0, True, False)
        step(1, True, False)
        lax.fori_loop(2, BPH - 1, lambda b, c: (step(b, True, True), c)[1], 0)
        step(BPH - 1, False, True)
        drain([out_cp(0, (BPH - 2) & (RING - 1))])
        drain([out_cp(0, (BPH - 1) & (RING - 1))])


@jax.jit
def kernel(row_indices, col_indices, row_table, col_table):
    ridx4 = jnp.transpose(
        row_indices.astype(jnp.int32).T.reshape(LT, 8, BT, 128), (0, 2, 1, 3))
    cidx4 = jnp.transpose(
        col_indices.astype(jnp.int32).T.reshape(LT, 8, BT, 128), (0, 2, 1, 3))
    k = pl.kernel(
        _body,
        mesh=plsc.VectorSubcoreMesh(core_axis_name="c", subcore_axis_name="s"),
        compiler_params=pltpu.CompilerParams(
            use_tc_tiling_on_sc=False, needs_layout_passes=False),
        out_type=jax.ShapeDtypeStruct((N, D), jnp.float32),
        scratch_types=[
            pltpu.VMEM((LT, 8, BPH), jnp.int32),
            pltpu.VMEM((LT, 8, BPH), jnp.int32),
            pltpu.VMEM((BPH, L), jnp.int32),
            pltpu.VMEM((BPH, L), jnp.int32),
            pltpu.VMEM((RING * L, D), jnp.float32),
            pltpu.VMEM_SHARED((V, D), jnp.float32),
            pltpu.VMEM_SHARED((V, D), jnp.float32),
            pltpu.SemaphoreType.DMA,
            pltpu.SemaphoreType.DMA,
            pltpu.SemaphoreType.DMA,
        ],
    )
    out = k(ridx4, cidx4, row_table, col_table)
    return out.reshape(B, L, D)


# speedup vs baseline: 3.4071x; 1.9628x over previous
import functools

import jax
import jax.numpy as jnp
from jax import lax
from jax.experimental import pallas as pl
from jax.experimental.pallas import tpu as pltpu
from jax.experimental.pallas import tpu_sc as plsc

B = 4096
L = 200
D = 64
V = 1000
NC = 2
NS = 16
NW = NC * NS
LT = L // 8
BT = B // 128
PH = 2
DH = D // PH


def _body(ridx4, cidx4, rowt_hbm, colt_hbm, out5,
          tstage, rowT_v, colT_v, rtile, ctile, obuf, semout):
    bt = lax.axis_index("s") * NC + lax.axis_index("c")
    lanes = lax.iota(jnp.int32, 16)

    def load_table_half(tbl_hbm, dstT, ph):
        pltpu.sync_copy(tbl_hbm.at[pl.ds(0, V), pl.ds(ph * DH, DH)], tstage)

        def trow(e, carry):
            ev = jnp.full((16,), e, jnp.int32)
            for g in range(DH // 16):
                v = tstage[e, pl.ds(g * 16, 16)]
                plsc.store_scatter(dstT, [lanes + (g * 16), ev], v)
            return carry
        lax.fori_loop(0, V, trow, 0)

    def out_cp(l, ph, p):
        return pltpu.make_async_copy(
            obuf.at[p],
            out5.at[l, pl.ds(ph * (DH // 8), DH // 8), bt], semout)

    for ph in range(PH):
        load_table_half(rowt_hbm, rowT_v, ph)
        load_table_half(colt_hbm, colT_v, ph)

        def ltile(tr, carry):
            pltpu.sync_copy(ridx4.at[tr, bt], rtile)
            pltpu.sync_copy(cidx4.at[tr, bt], ctile)

            def lrow(r, c2):
                l = tr * 8 + r
                p = r & 1

                @pl.when(l >= 2)
                def _drain_prev():
                    out_cp(0, 0, 0).wait()

                @plsc.parallel_loop(0, 8)
                def lane_group(g):
                    i16r = rtile[r, pl.ds(g * 16, 16)]
                    i16c = ctile[r, pl.ds(g * 16, 16)]
                    for dl in range(DH):
                        dv = jnp.full((16,), dl, jnp.int32)
                        v = (plsc.load_gather(rowT_v, [dv, i16r]) +
                             plsc.load_gather(colT_v, [dv, i16c]))
                        obuf[p, dl // 8, dl % 8, pl.ds(g * 16, 16)] = v

                out_cp(l, ph, p).start()
                return c2

            lax.fori_loop(0, 8, lrow, 0)
            return carry

        lax.fori_loop(0, LT, ltile, 0)
        out_cp(0, 0, 0).wait()
        out_cp(0, 0, 0).wait()


@jax.jit
def kernel(row_indices, col_indices, row_table, col_table):
    ridx4 = jnp.transpose(
        row_indices.astype(jnp.int32).T.reshape(LT, 8, BT, 128), (0, 2, 1, 3))
    cidx4 = jnp.transpose(
        col_indices.astype(jnp.int32).T.reshape(LT, 8, BT, 128), (0, 2, 1, 3))
    k = pl.kernel(
        _body,
        mesh=plsc.VectorSubcoreMesh(core_axis_name="c", subcore_axis_name="s"),
        compiler_params=pltpu.CompilerParams(
            use_tc_tiling_on_sc=False, needs_layout_passes=False),
        out_type=jax.ShapeDtypeStruct((L, D // 8, BT, 8, 128), jnp.float32),
        scratch_types=[
            pltpu.VMEM((V, DH), jnp.float32),
            pltpu.VMEM((DH, V), jnp.float32),
            pltpu.VMEM((DH, V), jnp.float32),
            pltpu.VMEM((8, 128), jnp.int32),
            pltpu.VMEM((8, 128), jnp.int32),
            pltpu.VMEM((2, DH // 8, 8, 128), jnp.float32),
            pltpu.SemaphoreType.DMA,
        ],
    )
    out5 = k(ridx4, cidx4, row_table, col_table)
    return out5.transpose(2, 4, 0, 1, 3).reshape(B, L, D)
